# baseline (speedup 1.0000x reference)
import functools

import jax
import jax.numpy as jnp
from jax import lax
from jax.experimental import pallas as pl
from jax.experimental.pallas import tpu as pltpu
from jax.experimental.pallas import tpu_sc as plsc

B = 8
NA = 128
NB = 256
L = 16
NVREG = NA // L
NEG = -1e20
LN2 = 0.6931471805599453


_LOGC = (-1.270023625799189, 1.6696577339751741,
         -0.4448658805190204, 0.050633374560152114)


def _softlog13(s):
    c = _LOGC
    p = c[3] * s + c[2]
    p = p * s + c[1]
    return p * s + c[0]


def _lse3(a, b, c, mw):
    m = jnp.maximum(jnp.maximum(a, b), c)
    s = (jnp.exp(a - m) + jnp.exp(b - m)) + jnp.exp(c - m)
    return _softlog13(s) + (m + mw)


def _dg(x, idx):
    return x.at[idx].get(mode="promise_in_bounds")


def _dtw_body(w_hbm, out_hbm, w_v, out_v):
    wid = lax.axis_index("s")

    @pl.when(jnp.logical_and(lax.axis_index("c") == 0, wid < B))
    def _():
        pltpu.sync_copy(w_hbm.at[wid], w_v)

        lanes = lax.iota(jnp.int32, L)
        neg = jnp.full((L,), NEG, jnp.float32)
        lane0 = lanes == 0
        sh_idx = jnp.maximum(lanes - 1, 0)
        hi_idx = jnp.full((L,), L - 1, jnp.int32)

        ivecs = [lanes + (v * L + 1) for v in range(NVREG)]
        c255 = [iv * (NB - 1) for iv in ivecs]

        d1 = [neg] * NVREG
        s1 = [neg] * NVREG
        s2 = [jnp.where(lane0, 0.0, NEG) if v == 0 else neg
              for v in range(NVREG)]

        def make_step(lo_g, hi_g, mask_kind):
            def step(k, carry):
                d1 = list(carry[:NVREG])
                s1 = list(carry[NVREG:2 * NVREG])
                s2 = list(carry[2 * NVREG:])
                new, news = list(d1), list(s1)
                for v in range(lo_g, hi_g):
                    wofs = c255[v] + (k - (NB + 1))
                    w = plsc.load_gather(w_v, [wofs])
                    val = _lse3(d1[v], s1[v], s2[v], w)
                    if mask_kind == "grow" and v == hi_g - 1:
                        msk = ivecs[v] <= k - 1
                    elif mask_kind == "shrink" and v == lo_g:
                        msk = ivecs[v] >= k - NB
                    else:
                        msk = None
                    if msk is None:
                        plsc.store_scatter(out_v, [wofs], val)
                    else:
                        val = jnp.where(msk, val, NEG)
                        plsc.store_scatter(out_v, [wofs], val, mask=msk)
                    carrier = neg if v == lo_g else _dg(new[v - 1], hi_idx)
                    news[v] = jnp.where(lane0, carrier, _dg(val, sh_idx))
                    new[v] = val
                if hi_g < NVREG:
                    news[hi_g] = jnp.where(
                        lane0, _dg(new[hi_g - 1], hi_idx), s1[hi_g])
                return tuple(new) + tuple(news) + tuple(s1)
            return step

        carry = tuple(d1) + tuple(s1) + tuple(s2)
        for g in range(1, NVREG + 1):
            carry = plsc.parallel_loop(
                L * (g - 1) + 2, L * g + 2, carry=carry,
                unroll=2)(make_step(0, g, "grow"))
        carry = plsc.parallel_loop(
            NA + 2, NB + 2, carry=carry,
            unroll=2)(make_step(0, NVREG, "full"))
        for h in range(NVREG):
            carry = plsc.parallel_loop(
                NB + 2 + L * h, min(NB + 2 + L * (h + 1), NA + NB + 1),
                carry=carry, unroll=2)(make_step(h, NVREG, "shrink"))
        pltpu.sync_copy(out_v, out_hbm.at[wid])


@jax.jit
def kernel(W):
    w_flat = W.reshape(B, NA * NB)
    mesh = plsc.VectorSubcoreMesh(core_axis_name="c", subcore_axis_name="s")
    out = pl.kernel(
        _dtw_body,
        mesh=mesh,
        compiler_params=pltpu.CompilerParams(needs_layout_passes=False),
        out_type=jax.ShapeDtypeStruct((B, NA * NB), jnp.float32),
        scratch_types=[
            pltpu.VMEM((NA * NB,), jnp.float32),
            pltpu.VMEM((NA * NB,), jnp.float32),
        ],
    )(w_flat)
    return out.reshape(B, NA, NB)


# speedup vs baseline: 1.0335x; 1.0335x over previous
import functools

import jax
import jax.numpy as jnp
from jax import lax
from jax.experimental import pallas as pl
from jax.experimental.pallas import tpu as pltpu
from jax.experimental.pallas import tpu_sc as plsc

B = 8
NA = 128
NB = 256
L = 16
NVREG = NA // L
NEG = -1e20
LN2 = 0.6931471805599453


_LOGC = (-1.270023625799189, 1.6696577339751741,
         -0.4448658805190204, 0.050633374560152114)


def _softlog13(s):
    c = _LOGC
    p = c[3] * s + c[2]
    p = p * s + c[1]
    return p * s + c[0]


def _lse3(a, b, c, mw):
    m = jnp.maximum(jnp.maximum(a, b), c)
    s = (jnp.exp(a - m) + jnp.exp(b - m)) + jnp.exp(c - m)
    return _softlog13(s) + (m + mw)


def _dg(x, idx):
    return x.at[idx].get(mode="promise_in_bounds")


def _dtw_body(w_hbm, out_hbm, w_v, out_v):
    wid = lax.axis_index("s")

    @pl.when(jnp.logical_and(lax.axis_index("c") == 0, wid < B))
    def _():
        pltpu.sync_copy(w_hbm.at[wid], w_v)

        lanes = lax.iota(jnp.int32, L)
        neg = jnp.full((L,), NEG, jnp.float32)
        lane0 = lanes == 0
        sh_idx = jnp.maximum(lanes - 1, 0)
        hi_idx = jnp.full((L,), L - 1, jnp.int32)

        ivecs = [lanes + (v * L + 1) for v in range(NVREG)]
        c255 = [iv * (NB - 1) for iv in ivecs]

        d1 = [neg] * NVREG
        s1 = [neg] * NVREG
        s2 = [jnp.where(lane0, 0.0, NEG) if v == 0 else neg
              for v in range(NVREG)]

        def make_step(lo_g, hi_g, mask_kind):
            def step(k, carry):
                d1 = list(carry[:NVREG])
                s1 = list(carry[NVREG:2 * NVREG])
                s2 = list(carry[2 * NVREG:])
                new, news = list(d1), list(s1)
                for v in range(lo_g, hi_g):
                    wofs = c255[v] + (k - (NB + 1))
                    w = plsc.load_gather(w_v, [wofs])
                    val = _lse3(d1[v], s1[v], s2[v], w)
                    if mask_kind == "grow" and v == hi_g - 1:
                        msk = ivecs[v] <= k - 1
                    elif mask_kind == "shrink" and v == lo_g:
                        msk = ivecs[v] >= k - NB
                    else:
                        msk = None
                    if msk is None:
                        plsc.store_scatter(out_v, [wofs], val)
                    else:
                        val = jnp.where(msk, val, NEG)
                        plsc.store_scatter(out_v, [wofs], val, mask=msk)
                    carrier = neg if v == lo_g else _dg(new[v - 1], hi_idx)
                    news[v] = jnp.where(lane0, carrier, _dg(val, sh_idx))
                    new[v] = val
                if hi_g < NVREG:
                    news[hi_g] = jnp.where(
                        lane0, _dg(new[hi_g - 1], hi_idx), s1[hi_g])
                return tuple(new) + tuple(news) + tuple(s1)
            return step

        carry = tuple(d1) + tuple(s1) + tuple(s2)
        for g in range(1, NVREG + 1):
            carry = plsc.parallel_loop(
                L * (g - 1) + 2, L * g + 2, carry=carry,
                unroll=2)(make_step(0, g, "grow"))
        carry = plsc.parallel_loop(
            NA + 2, NB + 2, carry=carry,
            unroll=2)(make_step(0, NVREG, "full"))
        for h in range(NVREG):
            carry = plsc.parallel_loop(
                NB + 2 + L * h, min(NB + 2 + L * (h + 1), NA + NB + 1),
                carry=carry, unroll=2)(make_step(h, NVREG, "shrink"))
        pltpu.sync_copy(out_v, out_hbm.at[wid])


@jax.jit
def kernel(W):
    w_flat = W.reshape(B, NA * NB)
    mesh = plsc.VectorSubcoreMesh(core_axis_name="c", subcore_axis_name="s",
                                  num_cores=1)
    out = pl.kernel(
        _dtw_body,
        mesh=mesh,
        compiler_params=pltpu.CompilerParams(needs_layout_passes=False),
        out_type=jax.ShapeDtypeStruct((B, NA * NB), jnp.float32),
        scratch_types=[
            pltpu.VMEM((NA * NB,), jnp.float32),
            pltpu.VMEM((NA * NB,), jnp.float32),
        ],
    )(w_flat)
    return out.reshape(B, NA, NB)


# speedup vs baseline: 1.0416x; 1.0078x over previous
import functools

import jax
import jax.numpy as jnp
from jax import lax
from jax.experimental import pallas as pl
from jax.experimental.pallas import tpu as pltpu
from jax.experimental.pallas import tpu_sc as plsc

B = 8
NA = 128
NB = 256
L = 16
NVREG = NA // L
NEG = -1e20
LN2 = 0.6931471805599453
LOG2E = 1.4426950408889634


_LOGC = (-1.270023625799189, 1.6696577339751741,
         -0.4448658805190204, 0.050633374560152114)


def _softlog13(s):
    c = _LOGC
    p = c[3] * s + c[2]
    p = p * s + c[1]
    return p * s + c[0]


def _lse3(a, b, c, mw):
    m = jnp.maximum(jnp.maximum(a, b), c)
    s = (jnp.exp(a - m) + jnp.exp(b - m)) + jnp.exp(c - m)
    return _softlog13(s) + (m + mw)


def _dg(x, idx):
    return x.at[idx].get(mode="promise_in_bounds")


def _dtw_body(w_hbm, out_hbm, w_v, out_v):
    wid = lax.axis_index("s")

    @pl.when(jnp.logical_and(lax.axis_index("c") == 0, wid < B))
    def _():
        pltpu.sync_copy(w_hbm.at[wid], w_v)

        lanes = lax.iota(jnp.int32, L)
        neg = jnp.full((L,), NEG, jnp.float32)
        lane0 = lanes == 0
        sh_idx = jnp.maximum(lanes - 1, 0)
        hi_idx = jnp.full((L,), L - 1, jnp.int32)

        ivecs = [lanes + (v * L + 1) for v in range(NVREG)]
        c255 = [iv * (NB - 1) for iv in ivecs]

        d1 = [neg] * NVREG
        s1 = [neg] * NVREG
        s2 = [jnp.where(lane0, 0.0, NEG) if v == 0 else neg
              for v in range(NVREG)]

        def make_step(lo_g, hi_g, mask_kind):
            def step(k, carry):
                d1 = list(carry[:NVREG])
                s1 = list(carry[NVREG:2 * NVREG])
                s2 = list(carry[2 * NVREG:])
                new, news = list(d1), list(s1)
                for v in range(lo_g, hi_g):
                    wofs = c255[v] + (k - (NB + 1))
                    w = plsc.load_gather(w_v, [wofs])
                    val = _lse3(d1[v], s1[v], s2[v], w)
                    if mask_kind == "grow" and v == hi_g - 1:
                        msk = ivecs[v] <= k - 1
                    elif mask_kind == "shrink" and v == lo_g:
                        msk = ivecs[v] >= k - NB
                    else:
                        msk = None
                    if msk is None:
                        plsc.store_scatter(out_v, [wofs], val)
                    else:
                        val = jnp.where(msk, val, NEG)
                        plsc.store_scatter(out_v, [wofs], val, mask=msk)
                    carrier = neg if v == lo_g else _dg(new[v - 1], hi_idx)
                    news[v] = jnp.where(lane0, carrier, _dg(val, sh_idx))
                    new[v] = val
                if hi_g < NVREG:
                    news[hi_g] = jnp.where(
                        lane0, _dg(new[hi_g - 1], hi_idx), s1[hi_g])
                return tuple(new) + tuple(news) + tuple(s1)
            return step

        carry = tuple(d1) + tuple(s1) + tuple(s2)
        for g in range(1, NVREG + 1):
            carry = plsc.parallel_loop(
                L * (g - 1) + 2, L * g + 2, carry=carry,
                unroll=2)(make_step(0, g, "grow"))
        carry = plsc.parallel_loop(
            NA + 2, NB + 2, carry=carry,
            unroll=4)(make_step(0, NVREG, "full"))
        for h in range(NVREG):
            carry = plsc.parallel_loop(
                NB + 2 + L * h, min(NB + 2 + L * (h + 1), NA + NB + 1),
                carry=carry, unroll=2)(make_step(h, NVREG, "shrink"))
        pltpu.sync_copy(out_v, out_hbm.at[wid])


@jax.jit
def kernel(W):
    w_flat = W.reshape(B, NA * NB)
    mesh = plsc.VectorSubcoreMesh(core_axis_name="c", subcore_axis_name="s",
                                  num_cores=1)
    out = pl.kernel(
        _dtw_body,
        mesh=mesh,
        compiler_params=pltpu.CompilerParams(needs_layout_passes=False),
        out_type=jax.ShapeDtypeStruct((B, NA * NB), jnp.float32),
        scratch_types=[
            pltpu.VMEM((NA * NB,), jnp.float32),
            pltpu.VMEM((NA * NB,), jnp.float32),
        ],
    )(w_flat)
    return out.reshape(B, NA, NB)
